# Initial kernel scaffold; baseline (speedup 1.0000x reference)
import jax
import jax.numpy as jnp
from jax.experimental import pallas as pl


def kernel(x, weight):
    raise NotImplementedError("write your pallas kernel here")


# speedup vs baseline: 8.1789x; 8.1789x over previous
import functools

import jax
import jax.numpy as jnp
from jax import lax
from jax.experimental import pallas as pl
from jax.experimental.pallas import tpu as pltpu
from jax.experimental.pallas import tpu_sc as plsc

D_MODEL = 128
B_TOTAL = 4096 * 200
NUM_CORES = 2
NUM_SUBCORES = 16
NW = NUM_CORES * NUM_SUBCORES
BPW = B_TOTAL // NW
CHUNK = 512
NCHUNK = BPW // CHUNK


@functools.partial(
    pl.kernel,
    mesh=plsc.VectorSubcoreMesh(core_axis_name="c", subcore_axis_name="s"),
    out_type=jax.ShapeDtypeStruct((B_TOTAL, D_MODEL), jnp.float32),
    scratch_types=[
        pltpu.VMEM((CHUNK,), jnp.int32),
        pltpu.VMEM((CHUNK, D_MODEL), jnp.float32),
        pltpu.SemaphoreType.DMA,
    ],
)
def _gather_kernel(idx_hbm, table_hbm, out_hbm, idx_v, rows_v, sem):
    wid = lax.axis_index("s") * NUM_CORES + lax.axis_index("c")
    base = wid * BPW

    def body(i, carry):
        off = base + i * CHUNK
        pltpu.sync_copy(idx_hbm.at[pl.ds(off, CHUNK)], idx_v)
        pltpu.async_copy(table_hbm.at[idx_v], rows_v, sem).wait()
        pltpu.sync_copy(rows_v, out_hbm.at[pl.ds(off, CHUNK)])
        return carry

    lax.fori_loop(0, NCHUNK, body, 0)


def kernel(x, weight):
    idx = x.reshape(-1).astype(jnp.int32)
    out = _gather_kernel(idx, weight)
    return out.reshape(x.shape + (weight.shape[1],))


# speedup vs baseline: 9.1793x; 1.1223x over previous
import functools

import jax
import jax.numpy as jnp
from jax import lax
from jax.experimental import pallas as pl
from jax.experimental.pallas import tpu as pltpu
from jax.experimental.pallas import tpu_sc as plsc

D_MODEL = 128
B_TOTAL = 4096 * 200
NUM_CORES = 2
NUM_SUBCORES = 16
NW = NUM_CORES * NUM_SUBCORES
BPW = B_TOTAL // NW
CHUNK = 200
NCHUNK = BPW // CHUNK
NBUF = 4
LOOKAHEAD = 2
P = NCHUNK // NBUF


@functools.partial(
    pl.kernel,
    mesh=plsc.VectorSubcoreMesh(core_axis_name="c", subcore_axis_name="s"),
    out_type=jax.ShapeDtypeStruct((B_TOTAL, D_MODEL), jnp.float32),
    scratch_types=[
        pltpu.VMEM((BPW,), jnp.int32),
        pltpu.VMEM((NBUF, CHUNK, D_MODEL), jnp.float32),
    ]
    + [pltpu.SemaphoreType.DMA] * (2 * NBUF),
)
def _gather_kernel(idx_hbm, table_hbm, out_hbm, idx_v, rows_v,
                   sg0, sg1, sg2, sg3, ss0, ss1, ss2, ss3):
    sg = (sg0, sg1, sg2, sg3)
    ss = (ss0, ss1, ss2, ss3)
    wid = lax.axis_index("s") * NUM_CORES + lax.axis_index("c")
    base = wid * BPW
    pltpu.sync_copy(idx_hbm.at[pl.ds(base, BPW)], idx_v)

    def issue_gather(j, u):
        pltpu.async_copy(
            table_hbm.at[idx_v.at[pl.ds(j * CHUNK, CHUNK)]],
            rows_v.at[u], sg[u])

    def wait_gather(u):
        pltpu.make_async_copy(
            table_hbm.at[pl.ds(0, CHUNK)], rows_v.at[u], sg[u]).wait()

    def issue_store(i, u):
        pltpu.async_copy(
            rows_v.at[u], out_hbm.at[pl.ds(base + i * CHUNK, CHUNK)], ss[u])

    def wait_store(u):
        pltpu.make_async_copy(
            table_hbm.at[pl.ds(0, CHUNK)], rows_v.at[u], ss[u]).wait()

    issue_gather(0, 0)
    issue_gather(1, 1)

    wait_gather(0); issue_store(0, 0); issue_gather(2, 2)
    wait_gather(1); issue_store(1, 1); issue_gather(3, 3)
    wait_gather(2); issue_store(2, 2); wait_store(0); issue_gather(4, 0)
    wait_gather(3); issue_store(3, 3); wait_store(1); issue_gather(5, 1)

    def body(p, carry):
        i0 = p * NBUF
        for u in range(NBUF):
            i = i0 + u
            bj = (u + LOOKAHEAD) % NBUF
            wait_gather(u)
            issue_store(i, u)
            wait_store(bj)
            issue_gather(i + LOOKAHEAD, bj)
        return carry

    lax.fori_loop(1, P - 1, body, 0)

    iN = NCHUNK - NBUF
    wait_gather(0); issue_store(iN + 0, 0); wait_store(2); issue_gather(iN + 2, 2)
    wait_gather(1); issue_store(iN + 1, 1); wait_store(3); issue_gather(iN + 3, 3)
    wait_gather(2); issue_store(iN + 2, 2)
    wait_gather(3); issue_store(iN + 3, 3)

    wait_store(0); wait_store(1); wait_store(2); wait_store(3)


def kernel(x, weight):
    idx = x.reshape(-1).astype(jnp.int32)
    out = _gather_kernel(idx, weight)
    return out.reshape(x.shape + (weight.shape[1],))


# speedup vs baseline: 9.1899x; 1.0012x over previous
import functools

import jax
import jax.numpy as jnp
from jax import lax
from jax.experimental import pallas as pl
from jax.experimental.pallas import tpu as pltpu
from jax.experimental.pallas import tpu_sc as plsc

D_MODEL = 128
B_TOTAL = 4096 * 200
NUM_CORES = 2
NUM_SUBCORES = 16
NW = NUM_CORES * NUM_SUBCORES
BPW = B_TOTAL // NW
CHUNK = 160
NCHUNK = BPW // CHUNK
NBUF = 5
LOOKAHEAD = 3
P = NCHUNK // NBUF


@functools.partial(
    pl.kernel,
    mesh=plsc.VectorSubcoreMesh(core_axis_name="c", subcore_axis_name="s"),
    out_type=jax.ShapeDtypeStruct((B_TOTAL, D_MODEL), jnp.float32),
    scratch_types=[
        pltpu.VMEM((BPW,), jnp.int32),
        pltpu.VMEM((NBUF, CHUNK, D_MODEL), jnp.float32),
    ]
    + [pltpu.SemaphoreType.DMA] * (2 * NBUF),
)
def _gather_kernel(idx_hbm, table_hbm, out_hbm, idx_v, rows_v, *sems):
    sg = sems[:NBUF]
    ss = sems[NBUF:]
    wid = lax.axis_index("s") * NUM_CORES + lax.axis_index("c")
    base = wid * BPW
    pltpu.sync_copy(idx_hbm.at[pl.ds(base, BPW)], idx_v)

    def issue_gather(j, u):
        pltpu.async_copy(
            table_hbm.at[idx_v.at[pl.ds(j * CHUNK, CHUNK)]],
            rows_v.at[u], sg[u])

    def wait_gather(u):
        pltpu.make_async_copy(
            table_hbm.at[pl.ds(0, CHUNK)], rows_v.at[u], sg[u]).wait()

    def issue_store(i, u):
        pltpu.async_copy(
            rows_v.at[u], out_hbm.at[pl.ds(base + i * CHUNK, CHUNK)], ss[u])

    def wait_store(u):
        pltpu.make_async_copy(
            table_hbm.at[pl.ds(0, CHUNK)], rows_v.at[u], ss[u]).wait()

    def step(i, u, with_gather=True, with_store_wait=True):
        wait_gather(u)
        issue_store(i, u)
        if with_gather:
            bj = (u + LOOKAHEAD) % NBUF
            if with_store_wait:
                wait_store(bj)
            issue_gather(i + LOOKAHEAD, bj)

    for u in range(LOOKAHEAD):
        issue_gather(u, u)

    for u in range(NBUF):
        step(u, u, with_store_wait=(u + LOOKAHEAD >= NBUF))

    def body(p, carry):
        i0 = p * NBUF
        for u in range(NBUF):
            step(i0 + u, u)
        return carry

    lax.fori_loop(1, P - 1, body, 0)

    iN = NCHUNK - NBUF
    for u in range(NBUF):
        step(iN + u, u, with_gather=(u + LOOKAHEAD < NBUF))

    for u in range(NBUF):
        wait_store(u)


def kernel(x, weight):
    idx = x.reshape(-1).astype(jnp.int32)
    out = _gather_kernel(idx, weight)
    return out.reshape(x.shape + (weight.shape[1],))
